# baseline (speedup 1.0000x reference)
import functools

import jax
import jax.numpy as jnp
from jax import lax
from jax.experimental import pallas as pl
from jax.experimental.pallas import tpu as pltpu
from jax.experimental.pallas import tpu_sc as plsc

_GATHER_DN = lax.GatherDimensionNumbers(
    offset_dims=(), collapsed_slice_dims=(0,), start_index_map=(0,))


def _xlane_take(t, idx):
    return lax.gather(t, idx[:, None], _GATHER_DN, (1,),
                      mode=lax.GatherScatterMode.PROMISE_IN_BOUNDS)


B = 16384
D = 64
NC = 2
NS = 16
NW = NC * NS
BPW = B // NW
L = 16


def _mf_body(u_hbm, i_hbm, w_hbm, b_hbm, ue_hbm, be_hbm, out_hbm,
             uidx_v, iidx_v, urows_v, irows_v, w_v, b_v, res_v,
             sem_u, sem_i):
    wid = lax.axis_index("s") * NC + lax.axis_index("c")
    base = wid * BPW

    pltpu.sync_copy(u_hbm.at[pl.ds(base, BPW)], uidx_v)
    pltpu.sync_copy(i_hbm.at[pl.ds(base, BPW)], iidx_v)
    cu = pltpu.async_copy(ue_hbm.at[uidx_v], urows_v, sem_u)
    ci = pltpu.async_copy(be_hbm.at[iidx_v], irows_v, sem_i)
    pltpu.sync_copy(w_hbm, w_v)
    pltpu.sync_copy(b_hbm, b_v)
    cu.wait()
    ci.wait()

    w0 = w_v[pl.ds(0, L)]
    w1 = w_v[pl.ds(L, L)]
    w2 = w_v[pl.ds(2 * L, L)]
    w3 = w_v[pl.ds(3 * L, L)]

    lane = lax.iota(jnp.int32, L)
    shuf = [lane ^ (1 << k) for k in range(4)]
    bias = b_v[...]

    def group(g, carry):
        r0 = g * L
        acc = jnp.zeros((L,), jnp.float32)
        for j in range(L):
            r = r0 + j
            t = (urows_v[r, pl.ds(0, L)] * irows_v[r, pl.ds(0, L)]) * w0
            t = t + (urows_v[r, pl.ds(L, L)] * irows_v[r, pl.ds(L, L)]) * w1
            t = t + (urows_v[r, pl.ds(2 * L, L)] * irows_v[r, pl.ds(2 * L, L)]) * w2
            t = t + (urows_v[r, pl.ds(3 * L, L)] * irows_v[r, pl.ds(3 * L, L)]) * w3
            for s in shuf:
                t = t + _xlane_take(t, s)
            acc = jnp.where(lane == j, t, acc)
        x = acc + bias
        res_v[pl.ds(r0, L)] = 10.0 / (1.0 + jnp.exp(-x))
        return carry

    lax.fori_loop(0, BPW // L, group, 0)

    pltpu.sync_copy(res_v, out_hbm.at[pl.ds(base, BPW)])


@jax.jit
def _mf(u, i, w_flat, b_vec, user_emb, book_emb):
    mesh = plsc.VectorSubcoreMesh(core_axis_name="c", subcore_axis_name="s")
    return pl.kernel(
        _mf_body,
        mesh=mesh,
        compiler_params=pltpu.CompilerParams(use_tc_tiling_on_sc=False),
        out_type=jax.ShapeDtypeStruct((B,), jnp.float32),
        scratch_types=[
            pltpu.VMEM((BPW,), jnp.int32),
            pltpu.VMEM((BPW,), jnp.int32),
            pltpu.VMEM((BPW, D), jnp.float32),
            pltpu.VMEM((BPW, D), jnp.float32),
            pltpu.VMEM((D,), jnp.float32),
            pltpu.VMEM((L,), jnp.float32),
            pltpu.VMEM((BPW,), jnp.float32),
            pltpu.SemaphoreType.DMA,
            pltpu.SemaphoreType.DMA,
        ],
    )(u, i, w_flat, b_vec, user_emb, book_emb)


def kernel(u, i, user_emb, book_emb, fc_w, fc_b):
    w_flat = fc_w.reshape(D)
    b_vec = jnp.broadcast_to(fc_b, (L,)).astype(jnp.float32)
    return _mf(u, i, w_flat, b_vec, user_emb, book_emb)


# speedup vs baseline: 1.5883x; 1.5883x over previous
import jax
import jax.numpy as jnp
from jax import lax
from jax.experimental import pallas as pl
from jax.experimental.pallas import tpu as pltpu
from jax.experimental.pallas import tpu_sc as plsc

B = 16384
D = 64
NC = 2
NS = 16
NW = NC * NS
L = 16
NV = 1000000
SLABW = 512
NSLAB = 61
SHIFT = 14
SROWS = 176
GROWS = B + 8
GRAVE = B


def _gather_body(u_hbm, i_hbm, ut_hbm, bt_hbm, utt_hbm, btt_hbm,
                 ug_hbm, ig_hbm,
                 allv, pk_v, slab_v, tail_v, stg_v, spos_v, scnt_s,
                 sem_a, sem_b, sem_s):
    hit_v = allv
    wid = lax.axis_index("s") * NC + lax.axis_index("c")
    nblk = jnp.where(wid < 4, 245, 244)
    blk_lo = wid * 244 + jnp.minimum(wid, 4)
    lo = blk_lo * 128
    hi = jnp.where(wid == NW - 1, NV, lo + nblk * 128)
    lane = lax.iota(jnp.int32, L)

    zero = jnp.zeros((L,), jnp.float32)
    grave_v = jnp.broadcast_to(jnp.int32(GRAVE), (L,))

    def clear_spos():
        for c in range(SROWS // L):
            spos_v[pl.ds(c * L, L)] = grave_v

    def flush(out_ref):
        copy = pltpu.make_async_copy(stg_v, out_ref.at[spos_v], sem_s)
        copy.start()
        copy.wait()
        clear_spos()
        scnt_s[0] = jnp.int32(0)

    for tbl in range(2):
        idx_hbm = u_hbm if tbl == 0 else i_hbm
        tab_hbm = ut_hbm if tbl == 0 else bt_hbm
        tail_hbm = utt_hbm if tbl == 0 else btt_hbm
        out_hbm = ug_hbm if tbl == 0 else ig_hbm

        pltpu.sync_copy(idx_hbm, allv)

        lo_v = jnp.broadcast_to(lo, (L,))
        hi_v = jnp.broadcast_to(hi, (L,))

        def scan_chunk(c, ptr):
            v = allv[pl.ds(c * L, L)]
            m = (v >= lo_v) & (v < hi_v)
            pv = lane + c * L
            packed = ((v - lo_v) << SHIFT) | pv
            plsc.store_compressed(pk_v.at[pl.ds(ptr, L)], packed, mask=m)
            return ptr + plsc.all_reduce_population_count(m)[0]

        n = lax.fori_loop(0, B // L, scan_chunk, jnp.int32(0))
        n_v = jnp.broadcast_to(n, (L,))
        nchunks = (n + (L - 1)) // L
        clear_spos()
        scnt_s[0] = jnp.int32(0)

        def process(slab_ref, cl, width):
            crel = cl - lo
            plo_v = jnp.broadcast_to(crel << SHIFT, (L,))
            phi_v = jnp.broadcast_to((crel + width) << SHIFT, (L,))

            def rescan(j, hp):
                pk = pk_v[pl.ds(j * L, L)]
                valid = (lane + j * L) < n_v
                mm = (pk >= plo_v) & (pk < phi_v) & valid
                plsc.store_compressed(hit_v.at[pl.ds(hp, L)], pk, mask=mm)
                return hp + plsc.all_reduce_population_count(mm)[0]

            h = lax.fori_loop(0, nchunks, rescan, jnp.int32(0))
            h_v = jnp.broadcast_to(h, (L,))
            trips = (h + (L - 1)) // L

            def extract(bt, carry):
                @pl.when(scnt_s[0] > SROWS - L)
                def _fl():
                    flush(out_hbm)

                sc = scnt_s[0]
                pk = hit_v[pl.ds(bt * L, L)]
                mloc = (pk >> SHIFT) - jnp.broadcast_to(crel, (L,))
                mpos = pk & ((1 << SHIFT) - 1)
                valid = (lane + bt * L) < h_v
                mloc = jnp.where(valid, mloc, 0)
                posx = jnp.where(valid, mpos, GRAVE)
                spos_v[pl.ds(sc, L)] = posx
                for j in range(L):
                    cvec = jnp.broadcast_to(mloc[j], (L,))
                    for k in range(D // L):
                        dvec = lane + k * L
                        g = plsc.load_gather(slab_ref, [dvec, cvec])
                        stg_v[sc + j, 0, pl.ds(k * L, L)] = g
                scnt_s[0] = sc + jnp.minimum(jnp.int32(L), h - bt * L)
                return carry

            lax.fori_loop(0, trips, extract, 0)

        def start_slab(slot, cl, sem):
            for ti in range(D // 8):
                pltpu.make_async_copy(
                    tab_hbm.at[pl.ds(8 * ti, 8), pl.ds(cl, SLABW)],
                    slab_v.at[slot, pl.ds(8 * ti, 8), :], sem,
                ).start()

        def wait_slab(slot, sem):
            pltpu.make_async_copy(
                tab_hbm.at[:, pl.ds(0, SLABW)], slab_v.at[slot], sem
            ).wait()

        start_slab(0, lo, sem_a)

        def pair(p, carry):
            s0 = 2 * p
            start_slab(1, lo + (s0 + 1) * SLABW, sem_b)
            wait_slab(0, sem_a)
            process(slab_v.at[0], lo + s0 * SLABW, SLABW)
            start_slab(0, lo + (s0 + 2) * SLABW, sem_a)
            wait_slab(1, sem_b)
            process(slab_v.at[1], lo + (s0 + 1) * SLABW, SLABW)
            return carry

        lax.fori_loop(0, NSLAB // 2, pair, 0)
        wait_slab(0, sem_a)
        process(slab_v.at[0], lo + (NSLAB - 1) * SLABW, SLABW)

        @pl.when(wid < 4)
        def _extra128():
            for ti in range(D // 8):
                pltpu.make_async_copy(
                    tab_hbm.at[pl.ds(8 * ti, 8), pl.ds(lo + NSLAB * SLABW, 128)],
                    slab_v.at[0, pl.ds(8 * ti, 8), pl.ds(0, 128)], sem_a,
                ).start()
            pltpu.make_async_copy(
                tab_hbm.at[:, pl.ds(0, 128)],
                slab_v.at[0, :, pl.ds(0, 128)], sem_a,
            ).wait()
            process(slab_v.at[0], lo + NSLAB * SLABW, 128)

        @pl.when(wid == NW - 1)
        def _extra64():
            pltpu.sync_copy(tail_hbm, tail_v)
            process(tail_v, NV - D, D)

        @pl.when(scnt_s[0] > 0)
        def _final_flush():
            flush(out_hbm)


@jax.jit
def _gather(u, i, ut, bt, utt, btt):
    mesh = plsc.VectorSubcoreMesh(core_axis_name="c", subcore_axis_name="s")
    return pl.kernel(
        _gather_body,
        mesh=mesh,
        compiler_params=pltpu.CompilerParams(needs_layout_passes=False),
        out_type=[
            jax.ShapeDtypeStruct((GROWS, 1, 128), jnp.float32),
            jax.ShapeDtypeStruct((GROWS, 1, 128), jnp.float32),
        ],
        scratch_types=[
            pltpu.VMEM((B,), jnp.int32),
            pltpu.VMEM((B + L,), jnp.int32),
            pltpu.VMEM((2, D, SLABW), jnp.float32),
            pltpu.VMEM((D, D), jnp.float32),
            pltpu.VMEM((SROWS, 1, 128), jnp.float32),
            pltpu.VMEM((SROWS,), jnp.int32),
            pltpu.SMEM((2,), jnp.int32),
            pltpu.SemaphoreType.DMA,
            pltpu.SemaphoreType.DMA,
            pltpu.SemaphoreType.DMA,
        ],
    )(u, i, ut, bt, utt, btt)


def _comb_body(b_ref, w_ref, ug_ref, ig_ref, o_ref):
    x = ug_ref[...][:, 0, :] * ig_ref[...][:, 0, :] * w_ref[...]
    s = jnp.sum(x, axis=1) + b_ref[0, 0]
    o_ref[...] = 10.0 / (1.0 + jnp.exp(-s))


@jax.jit
def _combine(b2, wpad, ug, ig):
    blk = 1024
    return pl.pallas_call(
        _comb_body,
        grid=(B // blk,),
        in_specs=[
            pl.BlockSpec(memory_space=pltpu.SMEM),
            pl.BlockSpec((1, 128), lambda k: (0, 0)),
            pl.BlockSpec((blk, 1, 128), lambda k: (k, 0, 0)),
            pl.BlockSpec((blk, 1, 128), lambda k: (k, 0, 0)),
        ],
        out_specs=pl.BlockSpec((blk,), lambda k: (k,)),
        out_shape=jax.ShapeDtypeStruct((B,), jnp.float32),
    )(b2, wpad, ug, ig)


def kernel(u, i, user_emb, book_emb, fc_w, fc_b):
    utt = user_emb[NV - D:, :].T
    btt = book_emb[NV - D:, :].T
    ug, ig = _gather(u, i, user_emb.T, book_emb.T, utt, btt)
    wpad = jnp.concatenate(
        [fc_w.reshape(1, D).astype(jnp.float32),
         jnp.zeros((1, 128 - D), jnp.float32)], axis=1)
    b2 = fc_b.reshape(1, 1).astype(jnp.float32)
    return _combine(b2, wpad, ug, ig)


# speedup vs baseline: 1.6155x; 1.0171x over previous
import jax
import jax.numpy as jnp
from jax import lax
from jax.experimental import pallas as pl
from jax.experimental.pallas import tpu as pltpu
from jax.experimental.pallas import tpu_sc as plsc

B = 16384
D = 64
NC = 2
NS = 16
NW = NC * NS
L = 16
NV = 1000000
SLABW = 512
NSLAB = 61
SHIFT = 14
HROWS = 80
GROWS = B + 8
GRAVE = B


def _gather_body(u_hbm, i_hbm, ut_hbm, bt_hbm, utt_hbm, btt_hbm,
                 ug_hbm, ig_hbm,
                 allv, pk_v, slab_v, tail_v, stg_v, spos_v, scnt_s,
                 sem_a, sem_b, sem_s0, sem_s1):
    hit_v = allv
    wid = lax.axis_index("s") * NC + lax.axis_index("c")
    nblk = jnp.where(wid < 4, 245, 244)
    blk_lo = wid * 244 + jnp.minimum(wid, 4)
    lo = blk_lo * 128
    hi = jnp.where(wid == NW - 1, NV, lo + nblk * 128)
    lane = lax.iota(jnp.int32, L)

    zero = jnp.zeros((L,), jnp.float32)
    grave_v = jnp.broadcast_to(jnp.int32(GRAVE), (L,))

    def clear_spos(q):
        for c in range(HROWS // L):
            spos_v[q, pl.ds(c * L, L)] = grave_v

    def flush(out_ref):
        par = scnt_s[1]

        @pl.when(par == 0)
        def _f0():
            pltpu.make_async_copy(
                stg_v.at[pl.ds(0, HROWS)], out_ref.at[spos_v.at[0]], sem_s0
            ).start()
            scnt_s[2] = jnp.int32(1)

            @pl.when(scnt_s[3] == 1)
            def _w1():
                pltpu.make_async_copy(
                    stg_v.at[pl.ds(HROWS, HROWS)], out_ref.at[spos_v.at[1]],
                    sem_s1,
                ).wait()

            scnt_s[3] = jnp.int32(0)
            clear_spos(1)

        @pl.when(par == 1)
        def _f1():
            pltpu.make_async_copy(
                stg_v.at[pl.ds(HROWS, HROWS)], out_ref.at[spos_v.at[1]], sem_s1
            ).start()
            scnt_s[3] = jnp.int32(1)

            @pl.when(scnt_s[2] == 1)
            def _w0():
                pltpu.make_async_copy(
                    stg_v.at[pl.ds(0, HROWS)], out_ref.at[spos_v.at[0]], sem_s0
                ).wait()

            scnt_s[2] = jnp.int32(0)
            clear_spos(0)

        scnt_s[0] = jnp.int32(0)
        scnt_s[1] = 1 - par

    def drain_all(out_ref):
        @pl.when(scnt_s[2] == 1)
        def _d0():
            pltpu.make_async_copy(
                stg_v.at[pl.ds(0, HROWS)], out_ref.at[spos_v.at[0]], sem_s0
            ).wait()

        @pl.when(scnt_s[3] == 1)
        def _d1():
            pltpu.make_async_copy(
                stg_v.at[pl.ds(HROWS, HROWS)], out_ref.at[spos_v.at[1]], sem_s1
            ).wait()

        scnt_s[2] = jnp.int32(0)
        scnt_s[3] = jnp.int32(0)

    for tbl in range(2):
        idx_hbm = u_hbm if tbl == 0 else i_hbm
        tab_hbm = ut_hbm if tbl == 0 else bt_hbm
        tail_hbm = utt_hbm if tbl == 0 else btt_hbm
        out_hbm = ug_hbm if tbl == 0 else ig_hbm

        pltpu.sync_copy(idx_hbm, allv)

        lo_v = jnp.broadcast_to(lo, (L,))
        hi_v = jnp.broadcast_to(hi, (L,))

        def scan_chunk(c, ptr):
            v = allv[pl.ds(c * L, L)]
            m = (v >= lo_v) & (v < hi_v)
            pv = lane + c * L
            packed = ((v - lo_v) << SHIFT) | pv
            plsc.store_compressed(pk_v.at[pl.ds(ptr, L)], packed, mask=m)
            return ptr + plsc.all_reduce_population_count(m)[0]

        n = lax.fori_loop(0, B // L, scan_chunk, jnp.int32(0))
        n_v = jnp.broadcast_to(n, (L,))
        nchunks = (n + (L - 1)) // L
        clear_spos(0)
        clear_spos(1)
        scnt_s[0] = jnp.int32(0)
        scnt_s[1] = jnp.int32(0)

        def process(slab_ref, cl, width):
            crel = cl - lo
            plo_v = jnp.broadcast_to(crel << SHIFT, (L,))
            phi_v = jnp.broadcast_to((crel + width) << SHIFT, (L,))

            def rescan(j, hp):
                pk = pk_v[pl.ds(j * L, L)]
                valid = (lane + j * L) < n_v
                mm = (pk >= plo_v) & (pk < phi_v) & valid
                plsc.store_compressed(hit_v.at[pl.ds(hp, L)], pk, mask=mm)
                return hp + plsc.all_reduce_population_count(mm)[0]

            h = lax.fori_loop(0, nchunks, rescan, jnp.int32(0))
            h_v = jnp.broadcast_to(h, (L,))
            trips = (h + (L - 1)) // L

            def extract(bt, carry):
                @pl.when(scnt_s[0] > HROWS - L)
                def _fl():
                    flush(out_hbm)

                sc = scnt_s[0] + scnt_s[1] * HROWS
                par = scnt_s[1]
                pk = hit_v[pl.ds(bt * L, L)]
                mloc = (pk >> SHIFT) - jnp.broadcast_to(crel, (L,))
                mpos = pk & ((1 << SHIFT) - 1)
                valid = (lane + bt * L) < h_v
                mloc = jnp.where(valid, mloc, 0)
                posx = jnp.where(valid, mpos, GRAVE)
                spos_v[par, pl.ds(scnt_s[0], L)] = posx
                for j in range(L):
                    cvec = jnp.broadcast_to(mloc[j], (L,))
                    for k in range(D // L):
                        dvec = lane + k * L
                        g = plsc.load_gather(slab_ref, [dvec, cvec])
                        stg_v[sc + j, 0, pl.ds(k * L, L)] = g
                scnt_s[0] = scnt_s[0] + jnp.minimum(jnp.int32(L), h - bt * L)
                return carry

            lax.fori_loop(0, trips, extract, 0)

        def start_slab(slot, cl, sem):
            for ti in range(D // 8):
                pltpu.make_async_copy(
                    tab_hbm.at[pl.ds(8 * ti, 8), pl.ds(cl, SLABW)],
                    slab_v.at[slot, pl.ds(8 * ti, 8), :], sem,
                ).start()

        def wait_slab(slot, sem):
            pltpu.make_async_copy(
                tab_hbm.at[:, pl.ds(0, SLABW)], slab_v.at[slot], sem
            ).wait()

        start_slab(0, lo, sem_a)

        def pair(p, carry):
            s0 = 2 * p
            start_slab(1, lo + (s0 + 1) * SLABW, sem_b)
            wait_slab(0, sem_a)
            process(slab_v.at[0], lo + s0 * SLABW, SLABW)
            start_slab(0, lo + (s0 + 2) * SLABW, sem_a)
            wait_slab(1, sem_b)
            process(slab_v.at[1], lo + (s0 + 1) * SLABW, SLABW)
            return carry

        lax.fori_loop(0, NSLAB // 2, pair, 0)
        wait_slab(0, sem_a)
        process(slab_v.at[0], lo + (NSLAB - 1) * SLABW, SLABW)

        @pl.when(wid < 4)
        def _extra128():
            for ti in range(D // 8):
                pltpu.make_async_copy(
                    tab_hbm.at[pl.ds(8 * ti, 8), pl.ds(lo + NSLAB * SLABW, 128)],
                    slab_v.at[0, pl.ds(8 * ti, 8), pl.ds(0, 128)], sem_a,
                ).start()
            pltpu.make_async_copy(
                tab_hbm.at[:, pl.ds(0, 128)],
                slab_v.at[0, :, pl.ds(0, 128)], sem_a,
            ).wait()
            process(slab_v.at[0], lo + NSLAB * SLABW, 128)

        @pl.when(wid == NW - 1)
        def _extra64():
            pltpu.sync_copy(tail_hbm, tail_v)
            process(tail_v, NV - D, D)

        @pl.when(scnt_s[0] > 0)
        def _final_flush():
            flush(out_hbm)

        drain_all(out_hbm)


@jax.jit
def _gather(u, i, ut, bt, utt, btt):
    mesh = plsc.VectorSubcoreMesh(core_axis_name="c", subcore_axis_name="s")
    return pl.kernel(
        _gather_body,
        mesh=mesh,
        compiler_params=pltpu.CompilerParams(needs_layout_passes=False),
        out_type=[
            jax.ShapeDtypeStruct((GROWS, 1, 128), jnp.float32),
            jax.ShapeDtypeStruct((GROWS, 1, 128), jnp.float32),
        ],
        scratch_types=[
            pltpu.VMEM((B,), jnp.int32),
            pltpu.VMEM((B + L,), jnp.int32),
            pltpu.VMEM((2, D, SLABW), jnp.float32),
            pltpu.VMEM((D, D), jnp.float32),
            pltpu.VMEM((2 * HROWS, 1, 128), jnp.float32),
            pltpu.VMEM((2, HROWS), jnp.int32),
            pltpu.SMEM((4,), jnp.int32),
            pltpu.SemaphoreType.DMA,
            pltpu.SemaphoreType.DMA,
            pltpu.SemaphoreType.DMA,
            pltpu.SemaphoreType.DMA,
        ],
    )(u, i, ut, bt, utt, btt)


def _comb_body(b_ref, w_ref, ug_ref, ig_ref, o_ref):
    x = ug_ref[...][:, 0, :] * ig_ref[...][:, 0, :] * w_ref[...]
    s = jnp.sum(x, axis=1) + b_ref[0, 0]
    o_ref[...] = 10.0 / (1.0 + jnp.exp(-s))


@jax.jit
def _combine(b2, wpad, ug, ig):
    blk = 1024
    return pl.pallas_call(
        _comb_body,
        grid=(B // blk,),
        in_specs=[
            pl.BlockSpec(memory_space=pltpu.SMEM),
            pl.BlockSpec((1, 128), lambda k: (0, 0)),
            pl.BlockSpec((blk, 1, 128), lambda k: (k, 0, 0)),
            pl.BlockSpec((blk, 1, 128), lambda k: (k, 0, 0)),
        ],
        out_specs=pl.BlockSpec((blk,), lambda k: (k,)),
        out_shape=jax.ShapeDtypeStruct((B,), jnp.float32),
    )(b2, wpad, ug, ig)


def kernel(u, i, user_emb, book_emb, fc_w, fc_b):
    utt = user_emb[NV - D:, :].T
    btt = book_emb[NV - D:, :].T
    ug, ig = _gather(u, i, user_emb.T, book_emb.T, utt, btt)
    wpad = jnp.concatenate(
        [fc_w.reshape(1, D).astype(jnp.float32),
         jnp.zeros((1, 128 - D), jnp.float32)], axis=1)
    b2 = fc_b.reshape(1, 1).astype(jnp.float32)
    return _combine(b2, wpad, ug, ig)


# speedup vs baseline: 1.6247x; 1.0057x over previous
import jax
import jax.numpy as jnp
from jax import lax
from jax.experimental import pallas as pl
from jax.experimental.pallas import tpu as pltpu
from jax.experimental.pallas import tpu_sc as plsc

B = 16384
D = 64
NC = 2
NS = 16
NW = NC * NS
L = 16
NV = 1000000
SLABW = 512
NSLAB = 61
SHIFT = 14
SROWS = 128
GROWS = B + 8
GRAVE = B


def _gather_body(u_hbm, i_hbm, ut_hbm, bt_hbm, utt_hbm, btt_hbm,
                 ug_hbm, ig_hbm,
                 allv, pk_v, slab_v, tail_v, stg_v, spos_v, scnt_s,
                 sem_a, sem_b, sem_s):
    hit_v = allv
    wid = lax.axis_index("s") * NC + lax.axis_index("c")
    nblk = jnp.where(wid < 4, 245, 244)
    blk_lo = wid * 244 + jnp.minimum(wid, 4)
    lo = blk_lo * 128
    hi = jnp.where(wid == NW - 1, NV, lo + nblk * 128)
    lane = lax.iota(jnp.int32, L)

    zero = jnp.zeros((L,), jnp.float32)
    grave_v = jnp.broadcast_to(jnp.int32(GRAVE), (L,))
    zerof = jnp.zeros((L,), jnp.float32)

    def zrow(r, carry):
        for k in range(D // L, 128 // L):
            stg_v[r, 0, pl.ds(k * L, L)] = zerof
        return carry

    lax.fori_loop(0, SROWS, zrow, 0)

    def clear_spos():
        for c in range(SROWS // L):
            spos_v[pl.ds(c * L, L)] = grave_v

    def flush(out_ref):
        copy = pltpu.make_async_copy(stg_v, out_ref.at[spos_v], sem_s)
        copy.start()
        copy.wait()
        clear_spos()
        scnt_s[0] = jnp.int32(0)

    for tbl in range(2):
        idx_hbm = u_hbm if tbl == 0 else i_hbm
        tab_hbm = ut_hbm if tbl == 0 else bt_hbm
        tail_hbm = utt_hbm if tbl == 0 else btt_hbm
        out_hbm = ug_hbm if tbl == 0 else ig_hbm

        pltpu.sync_copy(idx_hbm, allv)

        lo_v = jnp.broadcast_to(lo, (L,))
        hi_v = jnp.broadcast_to(hi, (L,))

        def scan_chunk(c, ptr):
            v = allv[pl.ds(c * L, L)]
            m = (v >= lo_v) & (v < hi_v)
            pv = lane + c * L
            packed = ((v - lo_v) << SHIFT) | pv
            plsc.store_compressed(pk_v.at[pl.ds(ptr, L)], packed, mask=m)
            return ptr + plsc.all_reduce_population_count(m)[0]

        n = lax.fori_loop(0, B // L, scan_chunk, jnp.int32(0))
        n_v = jnp.broadcast_to(n, (L,))
        nchunks = (n + (L - 1)) // L
        clear_spos()
        scnt_s[0] = jnp.int32(0)

        def process(slab_ref, cl, width):
            crel = cl - lo
            plo_v = jnp.broadcast_to(crel << SHIFT, (L,))
            phi_v = jnp.broadcast_to((crel + width) << SHIFT, (L,))

            def rescan(j, hp):
                pk = pk_v[pl.ds(j * L, L)]
                valid = (lane + j * L) < n_v
                mm = (pk >= plo_v) & (pk < phi_v) & valid
                plsc.store_compressed(hit_v.at[pl.ds(hp, L)], pk, mask=mm)
                return hp + plsc.all_reduce_population_count(mm)[0]

            h = lax.fori_loop(0, nchunks, rescan, jnp.int32(0))
            h_v = jnp.broadcast_to(h, (L,))
            trips = (h + (L - 1)) // L

            def extract(bt, carry):
                @pl.when(scnt_s[0] > SROWS - L)
                def _fl():
                    flush(out_hbm)

                sc = scnt_s[0]
                pk = hit_v[pl.ds(bt * L, L)]
                mloc = (pk >> SHIFT) - jnp.broadcast_to(crel, (L,))
                mpos = pk & ((1 << SHIFT) - 1)
                valid = (lane + bt * L) < h_v
                mloc = jnp.where(valid, mloc, 0)
                posx = jnp.where(valid, mpos, GRAVE)
                spos_v[pl.ds(sc, L)] = posx
                for j in range(L):
                    cvec = jnp.broadcast_to(mloc[j], (L,))
                    for k in range(D // L):
                        dvec = lane + k * L
                        g = plsc.load_gather(slab_ref, [dvec, cvec])
                        stg_v[sc + j, 0, pl.ds(k * L, L)] = g
                scnt_s[0] = sc + jnp.minimum(jnp.int32(L), h - bt * L)
                return carry

            lax.fori_loop(0, trips, extract, 0)

        def start_slab(slot, cl, sem):
            for ti in range(D // 8):
                pltpu.make_async_copy(
                    tab_hbm.at[pl.ds(8 * ti, 8), pl.ds(cl, SLABW)],
                    slab_v.at[slot, pl.ds(8 * ti, 8), :], sem,
                ).start()

        def wait_slab(slot, sem):
            pltpu.make_async_copy(
                tab_hbm.at[:, pl.ds(0, SLABW)], slab_v.at[slot], sem
            ).wait()

        start_slab(0, lo, sem_a)

        def pair(p, carry):
            s0 = 2 * p
            start_slab(1, lo + (s0 + 1) * SLABW, sem_b)
            wait_slab(0, sem_a)
            process(slab_v.at[0], lo + s0 * SLABW, SLABW)
            start_slab(0, lo + (s0 + 2) * SLABW, sem_a)
            wait_slab(1, sem_b)
            process(slab_v.at[1], lo + (s0 + 1) * SLABW, SLABW)
            return carry

        lax.fori_loop(0, NSLAB // 2, pair, 0)
        wait_slab(0, sem_a)
        process(slab_v.at[0], lo + (NSLAB - 1) * SLABW, SLABW)

        @pl.when(wid < 4)
        def _extra128():
            for ti in range(D // 8):
                pltpu.make_async_copy(
                    tab_hbm.at[pl.ds(8 * ti, 8), pl.ds(lo + NSLAB * SLABW, 128)],
                    slab_v.at[0, pl.ds(8 * ti, 8), pl.ds(0, 128)], sem_a,
                ).start()
            pltpu.make_async_copy(
                tab_hbm.at[:, pl.ds(0, 128)],
                slab_v.at[0, :, pl.ds(0, 128)], sem_a,
            ).wait()
            process(slab_v.at[0], lo + NSLAB * SLABW, 128)

        @pl.when(wid == NW - 1)
        def _extra64():
            pltpu.sync_copy(tail_hbm, tail_v)
            process(tail_v, NV - D, D)

        @pl.when(scnt_s[0] > 0)
        def _final_flush():
            flush(out_hbm)


@jax.jit
def _gather(u, i, ut, bt, utt, btt):
    mesh = plsc.VectorSubcoreMesh(core_axis_name="c", subcore_axis_name="s")
    return pl.kernel(
        _gather_body,
        mesh=mesh,
        compiler_params=pltpu.CompilerParams(needs_layout_passes=False),
        out_type=[
            jax.ShapeDtypeStruct((GROWS, 1, 128), jnp.float32),
            jax.ShapeDtypeStruct((GROWS, 1, 128), jnp.float32),
        ],
        scratch_types=[
            pltpu.VMEM((B,), jnp.int32),
            pltpu.VMEM((B + L,), jnp.int32),
            pltpu.VMEM((2, D, SLABW), jnp.float32),
            pltpu.VMEM((D, D), jnp.float32),
            pltpu.VMEM((SROWS, 1, 128), jnp.float32),
            pltpu.VMEM((SROWS,), jnp.int32),
            pltpu.SMEM((2,), jnp.int32),
            pltpu.SemaphoreType.DMA,
            pltpu.SemaphoreType.DMA,
            pltpu.SemaphoreType.DMA,
        ],
    )(u, i, ut, bt, utt, btt)


def _comb_body(b_ref, w_ref, ug_ref, ig_ref, o_ref):
    x = ug_ref[...][:, 0, :] * ig_ref[...][:, 0, :] * w_ref[...]
    s = jnp.sum(x, axis=1) + b_ref[0, 0]
    o_ref[...] = 10.0 / (1.0 + jnp.exp(-s))


@jax.jit
def _combine(b2, wpad, ug, ig):
    blk = 1024
    return pl.pallas_call(
        _comb_body,
        grid=(B // blk,),
        in_specs=[
            pl.BlockSpec(memory_space=pltpu.SMEM),
            pl.BlockSpec((1, 128), lambda k: (0, 0)),
            pl.BlockSpec((blk, 1, 128), lambda k: (k, 0, 0)),
            pl.BlockSpec((blk, 1, 128), lambda k: (k, 0, 0)),
        ],
        out_specs=pl.BlockSpec((blk,), lambda k: (k,)),
        out_shape=jax.ShapeDtypeStruct((B,), jnp.float32),
    )(b2, wpad, ug, ig)


def kernel(u, i, user_emb, book_emb, fc_w, fc_b):
    utt = user_emb[NV - D:, :].T
    btt = book_emb[NV - D:, :].T
    ug, ig = _gather(u, i, user_emb.T, book_emb.T, utt, btt)
    wpad = jnp.concatenate(
        [fc_w.reshape(1, D).astype(jnp.float32),
         jnp.zeros((1, 128 - D), jnp.float32)], axis=1)
    b2 = fc_b.reshape(1, 1).astype(jnp.float32)
    return _combine(b2, wpad, ug, ig)
